# baseline (speedup 1.0000x reference)
import functools

import jax
import jax.numpy as jnp
import numpy as np
from jax import lax
from jax.experimental import pallas as pl
from jax.experimental.pallas import tpu as pltpu
from jax.experimental.pallas import tpu_sc as plsc

N_NODES = 100000
N_EDGES = 1600000
D_IN = 27
H_DIM = 64

K_CHUNK = 128
GC = 5
GROUP_E = K_CHUNK * GC
N_TILES = 16
E_PAD = 1617920
EDGES_PER_TILE = E_PAD // N_TILES
N_GROUPS = EDGES_PER_TILE // GROUP_E
TABLE_ROWS = N_NODES + 64
ACC_ROWS = 100096
STRIPE = ACC_ROWS // N_TILES

_SOFTPLUS_BIAS = float(np.log(np.exp(1.0) - 1.0))


def _sc_segment_sum(tcat, src2, dst2):
    mesh = plsc.VectorSubcoreMesh(core_axis_name="c", subcore_axis_name="s")

    @functools.partial(
        pl.kernel,
        mesh=mesh,
        compiler_params=pltpu.CompilerParams(use_tc_tiling_on_sc=False),
        out_type=jax.ShapeDtypeStruct((2, ACC_ROWS, 16), jnp.float32),
        scratch_types=[
            pltpu.VMEM((GROUP_E,), jnp.int32),
            pltpu.VMEM((GROUP_E,), jnp.int32),
            pltpu.VMEM((GC, K_CHUNK), jnp.int32),
            pltpu.VMEM((GC, K_CHUNK), jnp.int32),
            pltpu.VMEM((GROUP_E, 16), jnp.float32),
            pltpu.VMEM((GROUP_E, 16), jnp.float32),
            pltpu.VMEM_SHARED((ACC_ROWS, 16), jnp.float32),
            pltpu.SemaphoreType.DMA,
            pltpu.SemaphoreType.DMA,
            pltpu.SemaphoreType.DMA,
            pltpu.SemaphoreType.DMA,
        ],
    )
    def k(tcat_hbm, src_hbm, dst_hbm, out_hbm,
          sbufA, sbufB, dbufA, dbufB, rowsA, rowsB, acc,
          semA, semB, semSA, semSB):
        cid = lax.axis_index("c")
        sid = lax.axis_index("s")
        edge_base = sid * EDGES_PER_TILE
        group_base = sid * N_GROUPS
        off_vec = jnp.full((16,), cid, jnp.int32)

        zv = jnp.zeros((16,), jnp.float32)

        def zloop(i, c):
            rowsA[i, :] = zv
            return c

        lax.fori_loop(0, GROUP_E, zloop, 0)
        for r in range(STRIPE // GROUP_E):
            pltpu.sync_copy(rowsA, acc.at[pl.ds(sid * STRIPE + r * GROUP_E, GROUP_E)])
        rem = STRIPE % GROUP_E
        pltpu.sync_copy(rowsA.at[pl.ds(0, rem)],
                        acc.at[pl.ds((sid + 1) * STRIPE - rem, rem)])
        plsc.subcore_barrier()

        def fire(g, sbuf, dbuf, rows, sem):
            pltpu.sync_copy(src_hbm.at[pl.ds(edge_base + g * GROUP_E, GROUP_E)], sbuf)
            pltpu.sync_copy(dst_hbm.at[group_base + g], dbuf)
            for q in range(GROUP_E // 16):
                sbuf[pl.ds(q * 16, 16)] = sbuf[pl.ds(q * 16, 16)] + off_vec
            for j in range(GC):
                pltpu.make_async_copy(
                    tcat_hbm.at[sbuf.at[pl.ds(j * K_CHUNK, K_CHUNK)]],
                    rows.at[pl.ds(j * K_CHUNK, K_CHUNK)],
                    sem,
                ).start()

        def drain(dbuf, rows, sem):
            pltpu.make_async_copy(tcat_hbm.at[pl.ds(0, GROUP_E)], rows, sem).wait()
            for j in range(GC):
                pltpu.sync_copy(
                    rows.at[pl.ds(j * K_CHUNK, K_CHUNK)],
                    acc.at[dbuf.at[j]],
                    add=True,
                )

        fire(0, sbufA, dbufA, rowsA, semA)

        def body(i, c):
            g = 2 * i
            fire(g + 1, sbufB, dbufB, rowsB, semB)
            drain(dbufA, rowsA, semA)

            @pl.when(i < N_GROUPS // 2 - 1)
            def _():
                fire(g + 2, sbufA, dbufA, rowsA, semA)

            drain(dbufB, rowsB, semB)
            return c

        lax.fori_loop(0, N_GROUPS // 2, body, 0)
        plsc.subcore_barrier()

        pltpu.sync_copy(
            acc.at[pl.ds(sid * STRIPE, STRIPE)],
            out_hbm.at[cid, pl.ds(sid * STRIPE, STRIPE)],
        )

    return k(tcat, src2, dst2)


def _tc_finish(xt, agg, W0, b0, W_rel, b_rel, W_root, W2, b2):
    TN = 4096
    grid = (pl.cdiv(N_NODES, TN),)
    prec = lax.Precision.HIGHEST
    bigprec = lax.Precision.DEFAULT

    def body(xt_ref, o_ref, w0_ref, b0_ref, wrel_ref, brel_ref, wroot_ref,
             w2_ref, b2_ref, loc_ref, scale_ref):
        i16 = lax.broadcasted_iota(jnp.int32, (16, 16), 0)
        j16 = lax.broadcasted_iota(jnp.int32, (16, 16), 1)
        eye16 = (i16 == j16).astype(jnp.float32)
        dnt = (((1,), (1,)), ((), ()))
        o0t = lax.dot_general(eye16, o_ref[0], dnt, precision=prec)
        o1t = lax.dot_general(eye16, o_ref[1], dnt, precision=prec)
        w0 = w0_ref[...]
        wrel = wrel_ref[...]
        wroot = wroot_ref[...]
        wr01 = jnp.dot(w0, wrel, precision=prec)
        wroot0 = jnp.dot(w0, wroot, precision=prec)
        b0v = b0_ref[...][None, :]
        cdeg = jnp.dot(b0v, wrel, precision=prec)
        cvec = (brel_ref[...] + jnp.dot(b0v, wroot, precision=prec)[0])[:, None]

        w1cat = jnp.concatenate(
            [wr01[16:27], cdeg, jnp.zeros((4, H_DIM), jnp.float32)], axis=0)
        dn = (((0,), (0,)), ((), ()))
        h2t = (lax.dot_general(wr01[:16], o0t, dn, precision=bigprec)
               + lax.dot_general(w1cat, o1t, dn, precision=bigprec)
               + lax.dot_general(wroot0, xt_ref[...], dn, precision=bigprec)
               + cvec)
        h2t = jnp.tanh(h2t)
        rt = lax.dot_general(w2_ref[...], h2t, dn, precision=bigprec)
        t = jnp.tanh(rt + b2_ref[...][:, None])
        loc_ref[...] = t[:8]
        z = t[8:] + _SOFTPLUS_BIAS
        scale_ref[...] = jnp.log1p(jnp.exp(z))

    full = lambda shape: pl.BlockSpec(shape, lambda t: (0,) * len(shape))
    loc, scale = pl.pallas_call(
        body,
        grid=grid,
        in_specs=[
            pl.BlockSpec((D_IN, TN), lambda t: (0, t)),
            pl.BlockSpec((2, TN, 16), lambda t: (0, t, 0)),
            full((D_IN, H_DIM)),
            full((H_DIM,)),
            full((H_DIM, H_DIM)),
            full((H_DIM,)),
            full((H_DIM, H_DIM)),
            full((H_DIM, 16)),
            full((16,)),
        ],
        out_specs=[
            pl.BlockSpec((8, TN), lambda t: (0, t)),
            pl.BlockSpec((8, TN), lambda t: (0, t)),
        ],
        out_shape=[
            jax.ShapeDtypeStruct((8, N_NODES), jnp.float32),
            jax.ShapeDtypeStruct((8, N_NODES), jnp.float32),
        ],
    )(xt, agg, W0, b0, W_rel, b_rel, W_root, W2, b2)
    return loc, scale


def kernel(x, edge_index, W0, b0, W_rel, b_rel, W_root, W2, b2):
    n = x.shape[0]
    xpad = jnp.concatenate(
        [x,
         jnp.ones((n, 1), jnp.float32),
         jnp.zeros((n, 4), jnp.float32)], axis=1)
    xpad = jnp.concatenate([xpad, jnp.zeros((64, 32), jnp.float32)], axis=0)
    tcat = xpad.reshape(2 * TABLE_ROWS, 16)

    pad = E_PAD - N_EDGES
    pidx = jnp.arange(pad, dtype=jnp.int32)
    src2 = jnp.concatenate([edge_index[0], n + (pidx % 64)]) * 2
    dst2 = jnp.concatenate([edge_index[1], pidx % 1024]).reshape(
        E_PAD // GROUP_E, GC, K_CHUNK)

    agg = _sc_segment_sum(tcat, src2, dst2)
    xt = x.T
    return _tc_finish(xt, agg, W0, b0, W_rel, b_rel, W_root, W2, b2)


# speedup vs baseline: 1.0981x; 1.0981x over previous
import functools

import jax
import jax.numpy as jnp
import numpy as np
from jax import lax
from jax.experimental import pallas as pl
from jax.experimental.pallas import tpu as pltpu
from jax.experimental.pallas import tpu_sc as plsc

N_NODES = 100000
N_EDGES = 1600000
D_IN = 27
H_DIM = 64

K_CHUNK = 128
GC = 5
GROUP_E = K_CHUNK * GC
N_TILES = 16
E_PAD = 1617920
EDGES_PER_TILE = E_PAD // N_TILES
N_GROUPS = EDGES_PER_TILE // GROUP_E
TABLE_ROWS = N_NODES + 64
ACC_ROWS = 100096
STRIPE = ACC_ROWS // N_TILES

_SOFTPLUS_BIAS = float(np.log(np.exp(1.0) - 1.0))


def _sc_segment_sum(tcat, src2, dst2):
    mesh = plsc.VectorSubcoreMesh(core_axis_name="c", subcore_axis_name="s")

    @functools.partial(
        pl.kernel,
        mesh=mesh,
        compiler_params=pltpu.CompilerParams(use_tc_tiling_on_sc=False),
        out_type=jax.ShapeDtypeStruct((2, ACC_ROWS, 16), jnp.float32),
        scratch_types=[
            pltpu.VMEM((GROUP_E,), jnp.int32),
            pltpu.VMEM((GROUP_E,), jnp.int32),
            pltpu.VMEM((GROUP_E,), jnp.int32),
            pltpu.VMEM((GROUP_E,), jnp.int32),
            pltpu.VMEM((GROUP_E, 16), jnp.float32),
            pltpu.VMEM((GROUP_E, 16), jnp.float32),
            pltpu.VMEM_SHARED((ACC_ROWS, 16), jnp.float32),
            pltpu.SemaphoreType.DMA,
            pltpu.SemaphoreType.DMA,
            pltpu.SemaphoreType.DMA,
            pltpu.SemaphoreType.DMA,
        ],
    )
    def k(tcat_hbm, src_hbm, dst_hbm, out_hbm,
          sbufA, sbufB, dbufA, dbufB, rowsA, rowsB, acc,
          semA, semB, semSA, semSB):
        cid = lax.axis_index("c")
        sid = lax.axis_index("s")
        edge_base = sid * EDGES_PER_TILE
        off_vec = jnp.full((16,), cid, jnp.int32)

        zv = jnp.zeros((16,), jnp.float32)

        def zloop(i, c):
            rowsA[i, :] = zv
            return c

        lax.fori_loop(0, GROUP_E, zloop, 0)
        for r in range(STRIPE // GROUP_E):
            pltpu.sync_copy(rowsA, acc.at[pl.ds(sid * STRIPE + r * GROUP_E, GROUP_E)])
        rem = STRIPE % GROUP_E
        pltpu.sync_copy(rowsA.at[pl.ds(0, rem)],
                        acc.at[pl.ds((sid + 1) * STRIPE - rem, rem)])
        plsc.subcore_barrier()

        def fire(g, sbuf, dbuf, rows, sem):
            pltpu.sync_copy(src_hbm.at[pl.ds(edge_base + g * GROUP_E, GROUP_E)], sbuf)
            pltpu.sync_copy(dst_hbm.at[pl.ds(edge_base + g * GROUP_E, GROUP_E)], dbuf)
            for q in range(GROUP_E // 16):
                sbuf[pl.ds(q * 16, 16)] = sbuf[pl.ds(q * 16, 16)] + off_vec
            for j in range(GC):
                pltpu.make_async_copy(
                    tcat_hbm.at[sbuf.at[pl.ds(j * K_CHUNK, K_CHUNK)]],
                    rows.at[pl.ds(j * K_CHUNK, K_CHUNK)],
                    sem,
                ).start()

        def drain(dbuf, rows, sem):
            pltpu.make_async_copy(tcat_hbm.at[pl.ds(0, GROUP_E)], rows, sem).wait()
            for j in range(GC):
                pltpu.sync_copy(
                    rows.at[pl.ds(j * K_CHUNK, K_CHUNK)],
                    acc.at[dbuf.at[pl.ds(j * K_CHUNK, K_CHUNK)]],
                    add=True,
                )

        fire(0, sbufA, dbufA, rowsA, semA)

        def body(i, c):
            g = 2 * i
            fire(g + 1, sbufB, dbufB, rowsB, semB)
            drain(dbufA, rowsA, semA)

            @pl.when(i < N_GROUPS // 2 - 1)
            def _():
                fire(g + 2, sbufA, dbufA, rowsA, semA)

            drain(dbufB, rowsB, semB)
            return c

        lax.fori_loop(0, N_GROUPS // 2, body, 0)
        plsc.subcore_barrier()

        pltpu.sync_copy(
            acc.at[pl.ds(sid * STRIPE, STRIPE)],
            out_hbm.at[cid, pl.ds(sid * STRIPE, STRIPE)],
        )

    return k(tcat, src2, dst2)


def _tc_finish(xt, aggt, W0, b0, W_rel, b_rel, W_root, W2, b2):
    TN = 4096
    grid = (pl.cdiv(N_NODES, TN),)
    prec = lax.Precision.HIGHEST
    bigprec = lax.Precision.DEFAULT

    def body(xt_ref, o_ref, w0_ref, b0_ref, wrel_ref, brel_ref, wroot_ref,
             w2_ref, b2_ref, loc_ref, scale_ref):
        w0 = w0_ref[...]
        wrel = wrel_ref[...]
        wroot = wroot_ref[...]
        wr01 = jnp.dot(w0, wrel, precision=prec)
        wroot0 = jnp.dot(w0, wroot, precision=prec)
        b0v = b0_ref[...][None, :]
        cdeg = jnp.dot(b0v, wrel, precision=prec)
        cvec = (brel_ref[...] + jnp.dot(b0v, wroot, precision=prec)[0])[:, None]

        w1cat = jnp.concatenate(
            [wr01[16:27], cdeg, jnp.zeros((4, H_DIM), jnp.float32)], axis=0)
        dn = (((0,), (0,)), ((), ()))
        h2t = (lax.dot_general(wr01[:16], o_ref[0], dn, precision=bigprec)
               + lax.dot_general(w1cat, o_ref[1], dn, precision=bigprec)
               + lax.dot_general(wroot0, xt_ref[...], dn, precision=bigprec)
               + cvec)
        h2t = jnp.tanh(h2t)
        rt = lax.dot_general(w2_ref[...], h2t, dn, precision=bigprec)
        t = jnp.tanh(rt + b2_ref[...][:, None])
        loc_ref[...] = t[:8]
        z = t[8:] + _SOFTPLUS_BIAS
        scale_ref[...] = jnp.log1p(jnp.exp(z))

    full = lambda shape: pl.BlockSpec(shape, lambda t: (0,) * len(shape))
    loc, scale = pl.pallas_call(
        body,
        grid=grid,
        in_specs=[
            pl.BlockSpec((D_IN, TN), lambda t: (0, t)),
            pl.BlockSpec((2, 16, TN), lambda t: (0, 0, t)),
            full((D_IN, H_DIM)),
            full((H_DIM,)),
            full((H_DIM, H_DIM)),
            full((H_DIM,)),
            full((H_DIM, H_DIM)),
            full((H_DIM, 16)),
            full((16,)),
        ],
        out_specs=[
            pl.BlockSpec((8, TN), lambda t: (0, t)),
            pl.BlockSpec((8, TN), lambda t: (0, t)),
        ],
        out_shape=[
            jax.ShapeDtypeStruct((8, N_NODES), jnp.float32),
            jax.ShapeDtypeStruct((8, N_NODES), jnp.float32),
        ],
    )(xt, aggt, W0, b0, W_rel, b_rel, W_root, W2, b2)
    return loc, scale


def kernel(x, edge_index, W0, b0, W_rel, b_rel, W_root, W2, b2):
    n = x.shape[0]
    xpad = jnp.concatenate(
        [x,
         jnp.ones((n, 1), jnp.float32),
         jnp.zeros((n, 4), jnp.float32)], axis=1)
    xpad = jnp.concatenate([xpad, jnp.zeros((64, 32), jnp.float32)], axis=0)
    tcat = xpad.reshape(2 * TABLE_ROWS, 16)

    pad = E_PAD - N_EDGES
    pidx = jnp.arange(pad, dtype=jnp.int32)
    src2 = jnp.concatenate([edge_index[0], n + (pidx % 64)]) * 2
    dst2 = jnp.concatenate([edge_index[1], pidx % 1024])

    agg = _sc_segment_sum(tcat, src2, dst2)
    aggt = jnp.transpose(agg, (0, 2, 1))
    xt = x.T
    return _tc_finish(xt, aggt, W0, b0, W_rel, b_rel, W_root, W2, b2)


# speedup vs baseline: 1.1457x; 1.0433x over previous
import functools

import jax
import jax.numpy as jnp
import numpy as np
from jax import lax
from jax.experimental import pallas as pl
from jax.experimental.pallas import tpu as pltpu
from jax.experimental.pallas import tpu_sc as plsc

N_NODES = 100000
N_EDGES = 1600000
D_IN = 27
H_DIM = 64

K_CHUNK = 128
GC = 6
GROUP_E = K_CHUNK * GC
N_TILES = 16
E_PAD = 1622016
EDGES_PER_TILE = E_PAD // N_TILES
N_GROUPS = EDGES_PER_TILE // GROUP_E
TABLE_ROWS = N_NODES + 64
ACC_ROWS = 100096
STRIPE = ACC_ROWS // N_TILES

_SOFTPLUS_BIAS = float(np.log(np.exp(1.0) - 1.0))


def _sc_segment_sum(tcat, src2, dst2):
    mesh = plsc.VectorSubcoreMesh(core_axis_name="c", subcore_axis_name="s")

    @functools.partial(
        pl.kernel,
        mesh=mesh,
        compiler_params=pltpu.CompilerParams(use_tc_tiling_on_sc=False),
        out_type=jax.ShapeDtypeStruct((2, ACC_ROWS, 16), jnp.float32),
        scratch_types=[
            pltpu.VMEM((GROUP_E,), jnp.int32),
            pltpu.VMEM((GROUP_E,), jnp.int32),
            pltpu.VMEM((GROUP_E,), jnp.int32),
            pltpu.VMEM((GROUP_E,), jnp.int32),
            pltpu.VMEM((GROUP_E, 16), jnp.float32),
            pltpu.VMEM((GROUP_E, 16), jnp.float32),
            pltpu.VMEM_SHARED((ACC_ROWS, 16), jnp.float32),
            pltpu.SemaphoreType.DMA,
            pltpu.SemaphoreType.DMA,
            pltpu.SemaphoreType.DMA,
            pltpu.SemaphoreType.DMA,
        ],
    )
    def k(tcat_hbm, src_hbm, dst_hbm, out_hbm,
          sbufA, sbufB, dbufA, dbufB, rowsA, rowsB, acc,
          semA, semB, semSA, semSB):
        cid = lax.axis_index("c")
        sid = lax.axis_index("s")
        edge_base = sid * EDGES_PER_TILE
        off_vec = jnp.full((16,), cid, jnp.int32)

        zv = jnp.zeros((16,), jnp.float32)

        def zloop(i, c):
            rowsA[i, :] = zv
            return c

        lax.fori_loop(0, GROUP_E, zloop, 0)
        for r in range(STRIPE // GROUP_E):
            pltpu.sync_copy(rowsA, acc.at[pl.ds(sid * STRIPE + r * GROUP_E, GROUP_E)])
        rem = STRIPE % GROUP_E
        pltpu.sync_copy(rowsA.at[pl.ds(0, rem)],
                        acc.at[pl.ds((sid + 1) * STRIPE - rem, rem)])
        plsc.subcore_barrier()

        def fire(g, sbuf, dbuf, rows, sem):
            pltpu.sync_copy(src_hbm.at[pl.ds(edge_base + g * GROUP_E, GROUP_E)], sbuf)
            pltpu.sync_copy(dst_hbm.at[pl.ds(edge_base + g * GROUP_E, GROUP_E)], dbuf)
            for q in range(GROUP_E // 16):
                sbuf[pl.ds(q * 16, 16)] = sbuf[pl.ds(q * 16, 16)] + off_vec
            for j in range(GC):
                pltpu.make_async_copy(
                    tcat_hbm.at[sbuf.at[pl.ds(j * K_CHUNK, K_CHUNK)]],
                    rows.at[pl.ds(j * K_CHUNK, K_CHUNK)],
                    sem,
                ).start()

        def drain(dbuf, rows, sem):
            pltpu.make_async_copy(tcat_hbm.at[pl.ds(0, GROUP_E)], rows, sem).wait()
            for j in range(GC):
                pltpu.sync_copy(
                    rows.at[pl.ds(j * K_CHUNK, K_CHUNK)],
                    acc.at[dbuf.at[pl.ds(j * K_CHUNK, K_CHUNK)]],
                    add=True,
                )

        fire(0, sbufA, dbufA, rowsA, semA)

        def body(i, c):
            g = 2 * i
            fire(g + 1, sbufB, dbufB, rowsB, semB)
            drain(dbufA, rowsA, semA)

            @pl.when(i < N_GROUPS // 2 - 1)
            def _():
                fire(g + 2, sbufA, dbufA, rowsA, semA)

            drain(dbufB, rowsB, semB)
            return c

        lax.fori_loop(0, N_GROUPS // 2, body, 0)
        plsc.subcore_barrier()

        pltpu.sync_copy(
            acc.at[pl.ds(sid * STRIPE, STRIPE)],
            out_hbm.at[cid, pl.ds(sid * STRIPE, STRIPE)],
        )

    return k(tcat, src2, dst2)


def _tc_finish(xt, aggt, W0, b0, W_rel, b_rel, W_root, W2, b2):
    TN = 4096
    grid = (pl.cdiv(N_NODES, TN),)
    prec = lax.Precision.HIGHEST
    bigprec = lax.Precision.DEFAULT

    def body(xt_ref, o_ref, w0_ref, b0_ref, wrel_ref, brel_ref, wroot_ref,
             w2_ref, b2_ref, loc_ref, scale_ref):
        w0 = w0_ref[...]
        wrel = wrel_ref[...]
        wroot = wroot_ref[...]
        wr01 = jnp.dot(w0, wrel, precision=prec)
        wroot0 = jnp.dot(w0, wroot, precision=prec)
        b0v = b0_ref[...][None, :]
        cdeg = jnp.dot(b0v, wrel, precision=prec)
        cvec = (brel_ref[...] + jnp.dot(b0v, wroot, precision=prec)[0])[:, None]

        w1cat = jnp.concatenate(
            [wr01[16:27], cdeg, jnp.zeros((4, H_DIM), jnp.float32)], axis=0)
        dn = (((0,), (0,)), ((), ()))
        h2t = (lax.dot_general(wr01[:16], o_ref[0], dn, precision=bigprec)
               + lax.dot_general(w1cat, o_ref[1], dn, precision=bigprec)
               + lax.dot_general(wroot0, xt_ref[...], dn, precision=bigprec)
               + cvec)
        h2t = jnp.tanh(h2t)
        rt = lax.dot_general(w2_ref[...], h2t, dn, precision=bigprec)
        t = jnp.tanh(rt + b2_ref[...][:, None])
        loc_ref[...] = t[:8]
        z = t[8:] + _SOFTPLUS_BIAS
        scale_ref[...] = jnp.log1p(jnp.exp(z))

    full = lambda shape: pl.BlockSpec(shape, lambda t: (0,) * len(shape))
    loc, scale = pl.pallas_call(
        body,
        grid=grid,
        in_specs=[
            pl.BlockSpec((D_IN, TN), lambda t: (0, t)),
            pl.BlockSpec((2, 16, TN), lambda t: (0, 0, t)),
            full((D_IN, H_DIM)),
            full((H_DIM,)),
            full((H_DIM, H_DIM)),
            full((H_DIM,)),
            full((H_DIM, H_DIM)),
            full((H_DIM, 16)),
            full((16,)),
        ],
        out_specs=[
            pl.BlockSpec((8, TN), lambda t: (0, t)),
            pl.BlockSpec((8, TN), lambda t: (0, t)),
        ],
        out_shape=[
            jax.ShapeDtypeStruct((8, N_NODES), jnp.float32),
            jax.ShapeDtypeStruct((8, N_NODES), jnp.float32),
        ],
    )(xt, aggt, W0, b0, W_rel, b_rel, W_root, W2, b2)
    return loc, scale


def kernel(x, edge_index, W0, b0, W_rel, b_rel, W_root, W2, b2):
    n = x.shape[0]
    xpad = jnp.concatenate(
        [x,
         jnp.ones((n, 1), jnp.float32),
         jnp.zeros((n, 4), jnp.float32)], axis=1)
    xpad = jnp.concatenate([xpad, jnp.zeros((64, 32), jnp.float32)], axis=0)
    tcat = xpad.reshape(2 * TABLE_ROWS, 16)

    pad = E_PAD - N_EDGES
    pidx = jnp.arange(pad, dtype=jnp.int32)
    src2 = jnp.concatenate([edge_index[0], n + (pidx % 64)]) * 2
    dst2 = jnp.concatenate([edge_index[1], pidx % 1024])

    agg = _sc_segment_sum(tcat, src2, dst2)
    aggt = jnp.transpose(agg, (0, 2, 1))
    xt = x.T
    return _tc_finish(xt, aggt, W0, b0, W_rel, b_rel, W_root, W2, b2)


# speedup vs baseline: 1.1620x; 1.0143x over previous
import functools

import jax
import jax.numpy as jnp
import numpy as np
from jax import lax
from jax.experimental import pallas as pl
from jax.experimental.pallas import tpu as pltpu
from jax.experimental.pallas import tpu_sc as plsc

N_NODES = 100000
N_EDGES = 1600000
D_IN = 27
H_DIM = 64

K_CHUNK = 128
GC = 6
GROUP_E = K_CHUNK * GC
N_TILES = 16
E_PAD = 1622016
EDGES_PER_TILE = E_PAD // N_TILES
N_GROUPS = EDGES_PER_TILE // GROUP_E
TABLE_ROWS = N_NODES + 64
ACC_ROWS = 100096
STRIPE = ACC_ROWS // N_TILES

_SOFTPLUS_BIAS = float(np.log(np.exp(1.0) - 1.0))


def _sc_segment_sum(tcat, src2, dst2):
    mesh = plsc.VectorSubcoreMesh(core_axis_name="c", subcore_axis_name="s")

    @functools.partial(
        pl.kernel,
        mesh=mesh,
        compiler_params=pltpu.CompilerParams(use_tc_tiling_on_sc=False),
        out_type=jax.ShapeDtypeStruct((2, ACC_ROWS, 16), jnp.float32),
        scratch_types=[
            pltpu.VMEM((GROUP_E,), jnp.int32),
            pltpu.VMEM((GROUP_E,), jnp.int32),
            pltpu.VMEM((GROUP_E,), jnp.int32),
            pltpu.VMEM((GROUP_E,), jnp.int32),
            pltpu.VMEM((GROUP_E, 16), jnp.float32),
            pltpu.VMEM((GROUP_E, 16), jnp.float32),
            pltpu.VMEM_SHARED((ACC_ROWS, 16), jnp.float32),
            pltpu.SemaphoreType.DMA,
            pltpu.SemaphoreType.DMA,
            pltpu.SemaphoreType.DMA,
            pltpu.SemaphoreType.DMA,
        ],
    )
    def k(tcat_hbm, src_hbm, dst_hbm, out_hbm,
          sbufA, sbufB, dbufA, dbufB, rowsA, rowsB, acc,
          semA, semB, semSA, semSB):
        cid = lax.axis_index("c")
        sid = lax.axis_index("s")
        edge_base = sid * EDGES_PER_TILE
        off_vec = jnp.full((16,), cid, jnp.int32)

        zv = jnp.zeros((16,), jnp.float32)

        def zloop(i, c):
            rowsA[i, :] = zv
            return c

        lax.fori_loop(0, GROUP_E, zloop, 0)
        for r in range(STRIPE // GROUP_E):
            pltpu.sync_copy(rowsA, acc.at[pl.ds(sid * STRIPE + r * GROUP_E, GROUP_E)])
        rem = STRIPE % GROUP_E
        pltpu.sync_copy(rowsA.at[pl.ds(0, rem)],
                        acc.at[pl.ds((sid + 1) * STRIPE - rem, rem)])
        plsc.subcore_barrier()

        def fire(g, sbuf, dbuf, rows, sem):
            pltpu.sync_copy(src_hbm.at[pl.ds(edge_base + g * GROUP_E, GROUP_E)], sbuf)
            pltpu.sync_copy(dst_hbm.at[pl.ds(edge_base + g * GROUP_E, GROUP_E)], dbuf)
            for q in range(GROUP_E // 16):
                sbuf[pl.ds(q * 16, 16)] = sbuf[pl.ds(q * 16, 16)] + off_vec
            for j in range(GC):
                pltpu.make_async_copy(
                    tcat_hbm.at[sbuf.at[pl.ds(j * K_CHUNK, K_CHUNK)]],
                    rows.at[pl.ds(j * K_CHUNK, K_CHUNK)],
                    sem,
                ).start()

        def drain(dbuf, rows, sem):
            pltpu.make_async_copy(tcat_hbm.at[pl.ds(0, GROUP_E)], rows, sem).wait()
            for j in range(GC):
                pltpu.sync_copy(
                    rows.at[pl.ds(j * K_CHUNK, K_CHUNK)],
                    acc.at[dbuf.at[pl.ds(j * K_CHUNK, K_CHUNK)]],
                    add=True,
                )

        fire(0, sbufA, dbufA, rowsA, semA)

        def body(i, c):
            g = 2 * i
            fire(g + 1, sbufB, dbufB, rowsB, semB)
            drain(dbufA, rowsA, semA)

            @pl.when(i < N_GROUPS // 2 - 1)
            def _():
                fire(g + 2, sbufA, dbufA, rowsA, semA)

            drain(dbufB, rowsB, semB)
            return c

        lax.fori_loop(0, N_GROUPS // 2, body, 0)
        plsc.subcore_barrier()

        pltpu.sync_copy(
            acc.at[pl.ds(sid * STRIPE, STRIPE)],
            out_hbm.at[cid, pl.ds(sid * STRIPE, STRIPE)],
        )

    return k(tcat, src2, dst2)


def _tc_finish(xt, aggt, W0, b0, W_rel, b_rel, W_root, W2, b2):
    TN = 8192
    grid = (pl.cdiv(N_NODES, TN),)
    prec = lax.Precision.HIGHEST
    bigprec = lax.Precision.DEFAULT

    def body(xt_ref, o_ref, w0_ref, b0_ref, wrel_ref, brel_ref, wroot_ref,
             w2_ref, b2_ref, loc_ref, scale_ref):
        w0 = w0_ref[...]
        wrel = wrel_ref[...]
        wroot = wroot_ref[...]
        wr01 = jnp.dot(w0, wrel, precision=prec)
        wroot0 = jnp.dot(w0, wroot, precision=prec)
        b0v = b0_ref[...][None, :]
        cdeg = jnp.dot(b0v, wrel, precision=prec)
        cvec = (brel_ref[...] + jnp.dot(b0v, wroot, precision=prec)[0])[:, None]

        w1cat = jnp.concatenate(
            [wr01[16:27], cdeg, jnp.zeros((4, H_DIM), jnp.float32)], axis=0)
        dn = (((0,), (0,)), ((), ()))
        h2t = (lax.dot_general(wr01[:16], o_ref[0], dn, precision=bigprec)
               + lax.dot_general(w1cat, o_ref[1], dn, precision=bigprec)
               + lax.dot_general(wroot0, xt_ref[...], dn, precision=bigprec)
               + cvec)
        h2t = jnp.tanh(h2t)
        rt = lax.dot_general(w2_ref[...], h2t, dn, precision=bigprec)
        t = jnp.tanh(rt + b2_ref[...][:, None])
        loc_ref[...] = t[:8]
        z = t[8:] + _SOFTPLUS_BIAS
        scale_ref[...] = jnp.log1p(jnp.exp(z))

    full = lambda shape: pl.BlockSpec(shape, lambda t: (0,) * len(shape))
    loc, scale = pl.pallas_call(
        body,
        grid=grid,
        in_specs=[
            pl.BlockSpec((D_IN, TN), lambda t: (0, t)),
            pl.BlockSpec((2, 16, TN), lambda t: (0, 0, t)),
            full((D_IN, H_DIM)),
            full((H_DIM,)),
            full((H_DIM, H_DIM)),
            full((H_DIM,)),
            full((H_DIM, H_DIM)),
            full((H_DIM, 16)),
            full((16,)),
        ],
        out_specs=[
            pl.BlockSpec((8, TN), lambda t: (0, t)),
            pl.BlockSpec((8, TN), lambda t: (0, t)),
        ],
        out_shape=[
            jax.ShapeDtypeStruct((8, N_NODES), jnp.float32),
            jax.ShapeDtypeStruct((8, N_NODES), jnp.float32),
        ],
    )(xt, aggt, W0, b0, W_rel, b_rel, W_root, W2, b2)
    return loc, scale


def kernel(x, edge_index, W0, b0, W_rel, b_rel, W_root, W2, b2):
    n = x.shape[0]
    xpad = jnp.concatenate(
        [x,
         jnp.ones((n, 1), jnp.float32),
         jnp.zeros((n, 4), jnp.float32)], axis=1)
    xpad = jnp.concatenate([xpad, jnp.zeros((64, 32), jnp.float32)], axis=0)
    tcat = xpad.reshape(2 * TABLE_ROWS, 16)

    pad = E_PAD - N_EDGES
    pidx = jnp.arange(pad, dtype=jnp.int32)
    src2 = jnp.concatenate([edge_index[0], n + (pidx % 64)]) * 2
    dst2 = jnp.concatenate([edge_index[1], pidx % 1024])

    agg = _sc_segment_sum(tcat, src2, dst2)
    aggt = jnp.transpose(agg, (0, 2, 1))
    xt = x.T
    return _tc_finish(xt, aggt, W0, b0, W_rel, b_rel, W_root, W2, b2)


# speedup vs baseline: 1.1685x; 1.0055x over previous
import functools

import jax
import jax.numpy as jnp
import numpy as np
from jax import lax
from jax.experimental import pallas as pl
from jax.experimental.pallas import tpu as pltpu
from jax.experimental.pallas import tpu_sc as plsc

N_NODES = 100000
N_EDGES = 1600000
D_IN = 27
H_DIM = 64

K_CHUNK = 128
GC = 6
GROUP_E = K_CHUNK * GC
N_TILES = 16
E_PAD = 1622016
EDGES_PER_TILE = E_PAD // N_TILES
N_GROUPS = EDGES_PER_TILE // GROUP_E
TABLE_ROWS = N_NODES + 64
ACC_ROWS = 100096
STRIPE = ACC_ROWS // N_TILES

_SOFTPLUS_BIAS = float(np.log(np.exp(1.0) - 1.0))


def _sc_segment_sum(tcat, src2, dst2):
    mesh = plsc.VectorSubcoreMesh(core_axis_name="c", subcore_axis_name="s")

    @functools.partial(
        pl.kernel,
        mesh=mesh,
        compiler_params=pltpu.CompilerParams(use_tc_tiling_on_sc=False),
        out_type=jax.ShapeDtypeStruct((2, ACC_ROWS, 16), jnp.float32),
        scratch_types=[
            pltpu.VMEM((GROUP_E,), jnp.int32),
            pltpu.VMEM((GROUP_E,), jnp.int32),
            pltpu.VMEM((GROUP_E,), jnp.int32),
            pltpu.VMEM((GROUP_E,), jnp.int32),
            pltpu.VMEM((GROUP_E, 16), jnp.float32),
            pltpu.VMEM((GROUP_E, 16), jnp.float32),
            pltpu.VMEM_SHARED((ACC_ROWS, 16), jnp.float32),
            pltpu.SemaphoreType.DMA,
            pltpu.SemaphoreType.DMA,
            pltpu.SemaphoreType.DMA,
            pltpu.SemaphoreType.DMA,
        ],
    )
    def k(tcat_hbm, src_hbm, dst_hbm, out_hbm,
          sbufA, sbufB, dbufA, dbufB, rowsA, rowsB, acc,
          semA, semB, semSA, semSB):
        cid = lax.axis_index("c")
        sid = lax.axis_index("s")
        edge_base = sid * EDGES_PER_TILE
        off_vec = jnp.full((16,), cid, jnp.int32)

        zv = jnp.zeros((16,), jnp.float32)

        def zloop(i, c):
            rowsA[i, :] = zv
            return c

        lax.fori_loop(0, GROUP_E, zloop, 0)
        for r in range(STRIPE // GROUP_E):
            pltpu.sync_copy(rowsA, acc.at[pl.ds(sid * STRIPE + r * GROUP_E, GROUP_E)])
        rem = STRIPE % GROUP_E
        pltpu.sync_copy(rowsA.at[pl.ds(0, rem)],
                        acc.at[pl.ds((sid + 1) * STRIPE - rem, rem)])
        plsc.subcore_barrier()

        def fire(g, sbuf, dbuf, rows, sem):
            pltpu.sync_copy(src_hbm.at[pl.ds(edge_base + g * GROUP_E, GROUP_E)], sbuf)
            pltpu.sync_copy(dst_hbm.at[pl.ds(edge_base + g * GROUP_E, GROUP_E)], dbuf)
            for q in range(GROUP_E // 16):
                sbuf[pl.ds(q * 16, 16)] = sbuf[pl.ds(q * 16, 16)] + off_vec
            for j in range(GC):
                pltpu.make_async_copy(
                    tcat_hbm.at[sbuf.at[pl.ds(j * K_CHUNK, K_CHUNK)]],
                    rows.at[pl.ds(j * K_CHUNK, K_CHUNK)],
                    sem,
                ).start()

        def drain(dbuf, rows, sem):
            pltpu.make_async_copy(tcat_hbm.at[pl.ds(0, GROUP_E)], rows, sem).wait()
            for j in range(GC):
                pltpu.sync_copy(
                    rows.at[pl.ds(j * K_CHUNK, K_CHUNK)],
                    acc.at[dbuf.at[pl.ds(j * K_CHUNK, K_CHUNK)]],
                    add=True,
                )

        fire(0, sbufA, dbufA, rowsA, semA)

        def body(i, c):
            g = 2 * i
            fire(g + 1, sbufB, dbufB, rowsB, semB)
            drain(dbufA, rowsA, semA)

            @pl.when(i < N_GROUPS // 2 - 1)
            def _():
                fire(g + 2, sbufA, dbufA, rowsA, semA)

            drain(dbufB, rowsB, semB)
            return c

        lax.fori_loop(0, N_GROUPS // 2, body, 0)
        plsc.subcore_barrier()

        pltpu.sync_copy(
            acc.at[pl.ds(sid * STRIPE, STRIPE)],
            out_hbm.at[cid, pl.ds(sid * STRIPE, STRIPE)],
        )

    return k(tcat, src2, dst2)


def _tc_finish(xt, aggt, W0, b0, W_rel, b_rel, W_root, W2, b2):
    TN = 16384
    grid = (pl.cdiv(N_NODES, TN),)
    prec = lax.Precision.HIGHEST
    bigprec = lax.Precision.DEFAULT

    def body(xt_ref, o_ref, w0_ref, b0_ref, wrel_ref, brel_ref, wroot_ref,
             w2_ref, b2_ref, loc_ref, scale_ref):
        w0 = w0_ref[...]
        wrel = wrel_ref[...]
        wroot = wroot_ref[...]
        wr01 = jnp.dot(w0, wrel, precision=prec)
        wroot0 = jnp.dot(w0, wroot, precision=prec)
        b0v = b0_ref[...][None, :]
        cdeg = jnp.dot(b0v, wrel, precision=prec)
        cvec = (brel_ref[...] + jnp.dot(b0v, wroot, precision=prec)[0])[:, None]

        w1cat = jnp.concatenate(
            [wr01[16:27], cdeg, jnp.zeros((4, H_DIM), jnp.float32)], axis=0)
        dn = (((0,), (0,)), ((), ()))
        h2t = (lax.dot_general(wr01[:16], o_ref[0], dn, precision=bigprec)
               + lax.dot_general(w1cat, o_ref[1], dn, precision=bigprec)
               + lax.dot_general(wroot0, xt_ref[...], dn, precision=bigprec)
               + cvec)
        h2t = jnp.tanh(h2t)
        rt = lax.dot_general(w2_ref[...], h2t, dn, precision=bigprec)
        t = jnp.tanh(rt + b2_ref[...][:, None])
        loc_ref[...] = t[:8]
        z = t[8:] + _SOFTPLUS_BIAS
        scale_ref[...] = jnp.log1p(jnp.exp(z))

    full = lambda shape: pl.BlockSpec(shape, lambda t: (0,) * len(shape))
    loc, scale = pl.pallas_call(
        body,
        grid=grid,
        in_specs=[
            pl.BlockSpec((D_IN, TN), lambda t: (0, t)),
            pl.BlockSpec((2, 16, TN), lambda t: (0, 0, t)),
            full((D_IN, H_DIM)),
            full((H_DIM,)),
            full((H_DIM, H_DIM)),
            full((H_DIM,)),
            full((H_DIM, H_DIM)),
            full((H_DIM, 16)),
            full((16,)),
        ],
        out_specs=[
            pl.BlockSpec((8, TN), lambda t: (0, t)),
            pl.BlockSpec((8, TN), lambda t: (0, t)),
        ],
        out_shape=[
            jax.ShapeDtypeStruct((8, N_NODES), jnp.float32),
            jax.ShapeDtypeStruct((8, N_NODES), jnp.float32),
        ],
    )(xt, aggt, W0, b0, W_rel, b_rel, W_root, W2, b2)
    return loc, scale


def kernel(x, edge_index, W0, b0, W_rel, b_rel, W_root, W2, b2):
    n = x.shape[0]
    xpad = jnp.concatenate(
        [x,
         jnp.ones((n, 1), jnp.float32),
         jnp.zeros((n, 4), jnp.float32)], axis=1)
    xpad = jnp.concatenate([xpad, jnp.zeros((64, 32), jnp.float32)], axis=0)
    tcat = xpad.reshape(2 * TABLE_ROWS, 16)

    pad = E_PAD - N_EDGES
    pidx = jnp.arange(pad, dtype=jnp.int32)
    src2 = jnp.concatenate([edge_index[0], n + (pidx % 64)]) * 2
    dst2 = jnp.concatenate([edge_index[1], pidx % 1024])

    agg = _sc_segment_sum(tcat, src2, dst2)
    aggt = jnp.transpose(agg, (0, 2, 1))
    xt = x.T
    return _tc_finish(xt, aggt, W0, b0, W_rel, b_rel, W_root, W2, b2)
